# baseline (speedup 1.0000x reference)
import functools

import jax
import jax.numpy as jnp
from jax import lax
from jax.experimental import pallas as pl
from jax.experimental.pallas import tpu as pltpu
from jax.experimental.pallas import tpu_sc as plsc

EPS = 1e-5
LOG_EPS = -11.512925464970229
NPTS = 100
TOP_N = 8
NEG = -1.0e30
IBIG = 2**30

NC, NS, L = 2, 16, 16
NW = NC * NS
B, D, M, R, M_OUT = 1024, 8, 5, 16384, 5
ROWS_PER_W = B // NW
MM = M + 1
NQ = MM ** 4
NGRP = R // L
KSUP = 4
NSUP = NGRP // KSUP


_GATHER_DNUMS = lax.GatherDimensionNumbers(
    offset_dims=(), collapsed_slice_dims=(0,), start_index_map=(0,))


def _vtake(v, idx):
    return lax.gather(v, idx[:, None], _GATHER_DNUMS, (1,),
                      mode=lax.GatherScatterMode.PROMISE_IN_BOUNDS)


def _butterfly(v, op):
    lane = lax.broadcasted_iota(jnp.int32, (L,), 0)
    for sh in (8, 4, 2, 1):
        v = op(v, _vtake(v, lane ^ sh))
    return v


def _allsum(v):
    return _butterfly(v, jnp.add)


def _allmax(v):
    return _butterfly(v, jnp.maximum)


def _allmin(v):
    return _butterfly(v, jnp.minimum)


def _sc_body(x_hbm, cf_hbm, sf_hbm, oc_hbm, os_hbm, qp_hbm, pc_hbm, out_hbm,
             xv, cfv, sfv, ocv, osv, qpv, pcv, logfv, pairv, qt0v, qt1v,
             stabv, outv):
    wid = lax.axis_index("s") * NC + lax.axis_index("c")
    base_row = wid * ROWS_PER_W

    pltpu.sync_copy(x_hbm.at[pl.ds(base_row * D, ROWS_PER_W * D)], xv)
    pltpu.sync_copy(cf_hbm, cfv)
    pltpu.sync_copy(sf_hbm, sfv)
    pltpu.sync_copy(oc_hbm, ocv)
    pltpu.sync_copy(os_hbm, osv)
    pltpu.sync_copy(qp_hbm, qpv)
    pltpu.sync_copy(pc_hbm, pcv)

    lane = lax.broadcasted_iota(jnp.int32, (L,), 0)
    seven = jnp.full((L,), 7, jnp.int32)

    s0vec = jnp.zeros((L,), jnp.float32)
    s1vec = jnp.zeros((L,), jnp.float32)
    ocvec = ocv[0:L]
    osvec = osv[0:L]
    for m in range(M_OUT):
        cm = _vtake(ocvec, jnp.full((L,), m, jnp.int32))
        sm = _vtake(osvec, jnp.full((L,), m, jnp.int32))
        inv = 0.5 / (sm * sm)
        e_acc = jnp.zeros((L,), jnp.float32)
        ue_acc = jnp.zeros((L,), jnp.float32)
        for g in range(7):
            p = lane + (16 * g)
            u = p.astype(jnp.float32) * (1.0 / (NPTS - 1))
            du = u - cm
            e = jnp.exp(-(du * du) * inv)
            e = jnp.where(p < NPTS, e, 0.0)
            e_acc = e_acc + e
            ue_acc = ue_acc + u * e
        s0vec = jnp.where(lane == m, _allsum(e_acc), s0vec)
        s1vec = jnp.where(lane == m, _allsum(ue_acc), s1vec)
    stabv[0:L] = s0vec
    stabv[L:2 * L] = s1vec

    kdiv = []
    kmod = []
    inv2s2 = []
    cvals = []
    for g in range(3):
        kg = lane + (16 * g)
        fi = kg // MM
        mj = kg - fi * MM
        kdiv.append(fi)
        kmod.append(mj)
        cidx = fi * M + jnp.minimum(mj, M - 1)
        cvals.append(plsc.load_gather(cfv, [cidx]))
        sg = plsc.load_gather(sfv, [cidx])
        inv2s2.append(0.5 / (sg * sg))

    qlane = lane

    def build_quads(g, carry):
        q = qlane + g * L
        a01 = q // 36
        a23 = q - a01 * 36
        v0 = (plsc.load_gather(pairv, [a01]) +
              plsc.load_gather(pairv, [48 + a23]))
        v1 = (plsc.load_gather(pairv, [96 + a01]) +
              plsc.load_gather(pairv, [144 + a23]))
        qt0v[pl.ds(g * L, L)] = v0
        qt1v[pl.ds(g * L, L)] = v1
        return carry

    def process_row(i, out_acc, half):
        b_local = half * 16 + i
        xbase = b_local * D

        for g in range(3):
            xg = plsc.load_gather(xv, [xbase + kdiv[g]])
            dx = xg - cvals[g]
            t = jnp.maximum(-(dx * dx) * inv2s2[g], LOG_EPS)
            t = jnp.where(kmod[g] == M, 0.0, t)
            logfv[g * L:(g + 1) * L] = t

        for p in range(4):
            for g in range(3):
                q = lane + (16 * g)
                a = jnp.minimum(q // MM, M)
                bq = jnp.minimum(q - (q // MM) * MM, M)
                va = plsc.load_gather(logfv, [(2 * p) * MM + a])
                vb = plsc.load_gather(logfv, [(2 * p + 1) * MM + bq])
                pairv[p * 48 + g * L:p * 48 + (g + 1) * L] = va + vb

        lax.fori_loop(0, NQ // L, build_quads, 0)

        def merge_group(args):
            topk, topv, v, off = args
            pcg = pcv[pl.ds(off, L)]
            sk, sv = plsc.sort_key_val(v, pcg, descending=False)
            w = topk >= sk
            mk = jnp.where(w, topk, sk)
            mv = jnp.where(w, topv, sv)
            nk, nv = plsc.sort_key_val(mk, mv, descending=True)
            return nk, nv

        def sg_body(sg, carry):
            topk, topv, th8 = carry
            vs = []
            rmax = None
            for j in range(KSUP):
                off = (sg * KSUP + j) * L
                qp = qpv[pl.ds(off, L)]
                q0 = lax.shift_right_logical(qp, 11)
                q1 = qp & 2047
                v = (plsc.load_gather(qt0v, [q0]) +
                     plsc.load_gather(qt1v, [q1]))
                vs.append(v)
                rmax = v if rmax is None else jnp.maximum(rmax, v)
            anyhit = plsc.all_reduce_population_count(rmax >= th8)[0] > 0

            def do_merges(carry2):
                topk2, topv2 = carry2
                for j in range(KSUP):
                    off = (sg * KSUP + j) * L
                    th = _vtake(topk2, seven)
                    hit = plsc.all_reduce_population_count(
                        vs[j] >= th)[0] > 0
                    topk2, topv2 = lax.cond(
                        hit, merge_group,
                        lambda a: (a[0], a[1]),
                        (topk2, topv2, vs[j], off))
                return topk2, topv2

            topk, topv = lax.cond(
                anyhit, do_merges, lambda c: c, (topk, topv))
            return topk, topv, _vtake(topk, seven)

        topk0 = jnp.full((L,), NEG, jnp.float32)
        topv0 = jnp.full((L,), IBIG, jnp.int32)
        topk, topv, _ = lax.fori_loop(
            0, NSUP, sg_body,
            (topk0, topv0, jnp.full((L,), NEG, jnp.float32)))

        selv = jnp.full((L,), NEG, jnp.float32)
        selp = jnp.full((L,), 0, jnp.int32)
        for n in range(TOP_N):
            mx = _allmax(topk)
            cand = jnp.where(topk == mx, topv, IBIG)
            sel = _allmin(cand)
            selv = jnp.where(lane == n, mx, selv)
            selp = jnp.where(lane == n, sel, selp)
            topk = jnp.where(topv == sel, NEG, topk)

        fv = jnp.exp(selv)
        cidx = selp & 7
        s0sel = plsc.load_gather(stabv, [cidx])
        s1sel = plsc.load_gather(stabv, [cidx + L])
        num = _allsum(fv * s1sel)
        den = _allsum(fv * s0sel) + EPS
        return jnp.where(lane == i, num / den, out_acc)

    for half in range(2):
        acc = lax.fori_loop(
            0, 16,
            lambda i, a, h=half: process_row(i, a, h),
            jnp.zeros((L,), jnp.float32))
        outv[half * L:(half + 1) * L] = acc

    pltpu.sync_copy(outv, out_hbm.at[pl.ds(base_row, ROWS_PER_W)])


def kernel(x, centers, sigmas, out_centers, out_sigmas, antecedents,
           consequents):
    a = jnp.where(antecedents < 0, M, antecedents).astype(jnp.int32)
    q0 = ((a[:, 0] * MM + a[:, 1]) * MM + a[:, 2]) * MM + a[:, 3]
    q1 = ((a[:, 4] * MM + a[:, 5]) * MM + a[:, 6]) * MM + a[:, 7]
    qpack = q0 * 2048 + q1
    ridx = jnp.arange(R, dtype=jnp.int32)
    pc = ridx * 8 + consequents.astype(jnp.int32)

    xf = x.reshape(B * D)
    cf = centers.reshape(D * M).astype(jnp.float32)
    cf = jnp.pad(cf, (0, 48 - D * M))
    sf = jnp.pad(sigmas.reshape(D * M).astype(jnp.float32),
                 (0, 48 - D * M), constant_values=1.0)
    oc = jnp.pad(out_centers.astype(jnp.float32), (0, 16 - M_OUT))
    osg = jnp.pad(out_sigmas.astype(jnp.float32), (0, 16 - M_OUT),
                  constant_values=1.0)

    mesh = plsc.VectorSubcoreMesh(core_axis_name="c", subcore_axis_name="s")
    f = pl.kernel(
        _sc_body,
        mesh=mesh,
        out_type=jax.ShapeDtypeStruct((B,), jnp.float32),
        scratch_types=[
            pltpu.VMEM((ROWS_PER_W * D,), jnp.float32),
            pltpu.VMEM((48,), jnp.float32),
            pltpu.VMEM((48,), jnp.float32),
            pltpu.VMEM((16,), jnp.float32),
            pltpu.VMEM((16,), jnp.float32),
            pltpu.VMEM((R,), jnp.int32),
            pltpu.VMEM((R,), jnp.int32),
            pltpu.VMEM((48,), jnp.float32),
            pltpu.VMEM((192,), jnp.float32),
            pltpu.VMEM((NQ,), jnp.float32),
            pltpu.VMEM((NQ,), jnp.float32),
            pltpu.VMEM((32,), jnp.float32),
            pltpu.VMEM((ROWS_PER_W,), jnp.float32),
        ],
        compiler_params=pltpu.CompilerParams(needs_layout_passes=False),
    )
    return f(xf, cf, sf, oc, osg, qpack, pc)


# speedup vs baseline: 1.0313x; 1.0313x over previous
import functools

import jax
import jax.numpy as jnp
from jax import lax
from jax.experimental import pallas as pl
from jax.experimental.pallas import tpu as pltpu
from jax.experimental.pallas import tpu_sc as plsc

EPS = 1e-5
LOG_EPS = -11.512925464970229
NPTS = 100
TOP_N = 8
NEG = -1.0e30
IBIG = 2**30

NC, NS, L = 2, 16, 16
NW = NC * NS
B, D, M, R, M_OUT = 1024, 8, 5, 16384, 5
ROWS_PER_W = B // NW
MM = M + 1
NQ = MM ** 4
NGRP = R // L
KSUP = 8
NSUP = NGRP // KSUP


_GATHER_DNUMS = lax.GatherDimensionNumbers(
    offset_dims=(), collapsed_slice_dims=(0,), start_index_map=(0,))


def _vtake(v, idx):
    return lax.gather(v, idx[:, None], _GATHER_DNUMS, (1,),
                      mode=lax.GatherScatterMode.PROMISE_IN_BOUNDS)


def _butterfly(v, op):
    lane = lax.broadcasted_iota(jnp.int32, (L,), 0)
    for sh in (8, 4, 2, 1):
        v = op(v, _vtake(v, lane ^ sh))
    return v


def _allsum(v):
    return _butterfly(v, jnp.add)


def _allmax(v):
    return _butterfly(v, jnp.maximum)


def _allmin(v):
    return _butterfly(v, jnp.minimum)


def _sc_body(x_hbm, cf_hbm, sf_hbm, oc_hbm, os_hbm, qp_hbm, pc_hbm, out_hbm,
             xv, cfv, sfv, ocv, osv, qpv, pcv, logfv, pairv, qt0v, qt1v,
             stabv, outv):
    wid = lax.axis_index("s") * NC + lax.axis_index("c")
    base_row = wid * ROWS_PER_W

    pltpu.sync_copy(x_hbm.at[pl.ds(base_row * D, ROWS_PER_W * D)], xv)
    pltpu.sync_copy(cf_hbm, cfv)
    pltpu.sync_copy(sf_hbm, sfv)
    pltpu.sync_copy(oc_hbm, ocv)
    pltpu.sync_copy(os_hbm, osv)
    pltpu.sync_copy(qp_hbm, qpv)
    pltpu.sync_copy(pc_hbm, pcv)

    lane = lax.broadcasted_iota(jnp.int32, (L,), 0)
    seven = jnp.full((L,), 7, jnp.int32)

    s0vec = jnp.zeros((L,), jnp.float32)
    s1vec = jnp.zeros((L,), jnp.float32)
    ocvec = ocv[0:L]
    osvec = osv[0:L]
    for m in range(M_OUT):
        cm = _vtake(ocvec, jnp.full((L,), m, jnp.int32))
        sm = _vtake(osvec, jnp.full((L,), m, jnp.int32))
        inv = 0.5 / (sm * sm)
        e_acc = jnp.zeros((L,), jnp.float32)
        ue_acc = jnp.zeros((L,), jnp.float32)
        for g in range(7):
            p = lane + (16 * g)
            u = p.astype(jnp.float32) * (1.0 / (NPTS - 1))
            du = u - cm
            e = jnp.exp(-(du * du) * inv)
            e = jnp.where(p < NPTS, e, 0.0)
            e_acc = e_acc + e
            ue_acc = ue_acc + u * e
        s0vec = jnp.where(lane == m, _allsum(e_acc), s0vec)
        s1vec = jnp.where(lane == m, _allsum(ue_acc), s1vec)
    stabv[0:L] = s0vec
    stabv[L:2 * L] = s1vec

    kdiv = []
    kmod = []
    inv2s2 = []
    cvals = []
    for g in range(3):
        kg = lane + (16 * g)
        fi = kg // MM
        mj = kg - fi * MM
        kdiv.append(fi)
        kmod.append(mj)
        cidx = fi * M + jnp.minimum(mj, M - 1)
        cvals.append(plsc.load_gather(cfv, [cidx]))
        sg = plsc.load_gather(sfv, [cidx])
        inv2s2.append(0.5 / (sg * sg))

    qlane = lane

    def build_quads(g, carry):
        q = qlane + g * L
        a01 = q // 36
        a23 = q - a01 * 36
        v0 = (plsc.load_gather(pairv, [a01]) +
              plsc.load_gather(pairv, [48 + a23]))
        v1 = (plsc.load_gather(pairv, [96 + a01]) +
              plsc.load_gather(pairv, [144 + a23]))
        qt0v[pl.ds(g * L, L)] = v0
        qt1v[pl.ds(g * L, L)] = v1
        return carry

    def process_row(i, out_acc, half):
        b_local = half * 16 + i
        xbase = b_local * D

        for g in range(3):
            xg = plsc.load_gather(xv, [xbase + kdiv[g]])
            dx = xg - cvals[g]
            t = jnp.maximum(-(dx * dx) * inv2s2[g], LOG_EPS)
            t = jnp.where(kmod[g] == M, 0.0, t)
            logfv[g * L:(g + 1) * L] = t

        for p in range(4):
            for g in range(3):
                q = lane + (16 * g)
                a = jnp.minimum(q // MM, M)
                bq = jnp.minimum(q - (q // MM) * MM, M)
                va = plsc.load_gather(logfv, [(2 * p) * MM + a])
                vb = plsc.load_gather(logfv, [(2 * p + 1) * MM + bq])
                pairv[p * 48 + g * L:p * 48 + (g + 1) * L] = va + vb

        lax.fori_loop(0, NQ // L, build_quads, 0)

        def merge_group(args):
            topk, topv, v, off = args
            pcg = pcv[pl.ds(off, L)]
            sk, sv = plsc.sort_key_val(v, pcg, descending=False)
            w = topk >= sk
            mk = jnp.where(w, topk, sk)
            mv = jnp.where(w, topv, sv)
            nk, nv = plsc.sort_key_val(mk, mv, descending=True)
            return nk, nv

        def sg_body(sg, carry):
            topk, topv, th8 = carry
            vs = []
            rmax = None
            for j in range(KSUP):
                off = (sg * KSUP + j) * L
                qp = qpv[pl.ds(off, L)]
                q0 = lax.shift_right_logical(qp, 11)
                q1 = qp & 2047
                v = (plsc.load_gather(qt0v, [q0]) +
                     plsc.load_gather(qt1v, [q1]))
                vs.append(v)
                rmax = v if rmax is None else jnp.maximum(rmax, v)
            anyhit = plsc.all_reduce_population_count(rmax >= th8)[0] > 0

            def do_merges(carry2):
                topk2, topv2 = carry2
                for j in range(KSUP):
                    off = (sg * KSUP + j) * L
                    th = _vtake(topk2, seven)
                    hit = plsc.all_reduce_population_count(
                        vs[j] >= th)[0] > 0
                    topk2, topv2 = lax.cond(
                        hit, merge_group,
                        lambda a: (a[0], a[1]),
                        (topk2, topv2, vs[j], off))
                return topk2, topv2

            topk, topv = lax.cond(
                anyhit, do_merges, lambda c: c, (topk, topv))
            return topk, topv, _vtake(topk, seven)

        topk0 = jnp.full((L,), NEG, jnp.float32)
        topv0 = jnp.full((L,), IBIG, jnp.int32)
        topk, topv, _ = lax.fori_loop(
            0, NSUP, sg_body,
            (topk0, topv0, jnp.full((L,), NEG, jnp.float32)))

        selv = jnp.full((L,), NEG, jnp.float32)
        selp = jnp.full((L,), 0, jnp.int32)
        for n in range(TOP_N):
            mx = _allmax(topk)
            cand = jnp.where(topk == mx, topv, IBIG)
            sel = _allmin(cand)
            selv = jnp.where(lane == n, mx, selv)
            selp = jnp.where(lane == n, sel, selp)
            topk = jnp.where(topv == sel, NEG, topk)

        fv = jnp.exp(selv)
        cidx = selp & 7
        s0sel = plsc.load_gather(stabv, [cidx])
        s1sel = plsc.load_gather(stabv, [cidx + L])
        num = _allsum(fv * s1sel)
        den = _allsum(fv * s0sel) + EPS
        return jnp.where(lane == i, num / den, out_acc)

    for half in range(2):
        acc = lax.fori_loop(
            0, 16,
            lambda i, a, h=half: process_row(i, a, h),
            jnp.zeros((L,), jnp.float32))
        outv[half * L:(half + 1) * L] = acc

    pltpu.sync_copy(outv, out_hbm.at[pl.ds(base_row, ROWS_PER_W)])


def kernel(x, centers, sigmas, out_centers, out_sigmas, antecedents,
           consequents):
    a = jnp.where(antecedents < 0, M, antecedents).astype(jnp.int32)
    q0 = ((a[:, 0] * MM + a[:, 1]) * MM + a[:, 2]) * MM + a[:, 3]
    q1 = ((a[:, 4] * MM + a[:, 5]) * MM + a[:, 6]) * MM + a[:, 7]
    qpack = q0 * 2048 + q1
    ridx = jnp.arange(R, dtype=jnp.int32)
    pc = ridx * 8 + consequents.astype(jnp.int32)

    xf = x.reshape(B * D)
    cf = centers.reshape(D * M).astype(jnp.float32)
    cf = jnp.pad(cf, (0, 48 - D * M))
    sf = jnp.pad(sigmas.reshape(D * M).astype(jnp.float32),
                 (0, 48 - D * M), constant_values=1.0)
    oc = jnp.pad(out_centers.astype(jnp.float32), (0, 16 - M_OUT))
    osg = jnp.pad(out_sigmas.astype(jnp.float32), (0, 16 - M_OUT),
                  constant_values=1.0)

    mesh = plsc.VectorSubcoreMesh(core_axis_name="c", subcore_axis_name="s")
    f = pl.kernel(
        _sc_body,
        mesh=mesh,
        out_type=jax.ShapeDtypeStruct((B,), jnp.float32),
        scratch_types=[
            pltpu.VMEM((ROWS_PER_W * D,), jnp.float32),
            pltpu.VMEM((48,), jnp.float32),
            pltpu.VMEM((48,), jnp.float32),
            pltpu.VMEM((16,), jnp.float32),
            pltpu.VMEM((16,), jnp.float32),
            pltpu.VMEM((R,), jnp.int32),
            pltpu.VMEM((R,), jnp.int32),
            pltpu.VMEM((48,), jnp.float32),
            pltpu.VMEM((192,), jnp.float32),
            pltpu.VMEM((NQ,), jnp.float32),
            pltpu.VMEM((NQ,), jnp.float32),
            pltpu.VMEM((32,), jnp.float32),
            pltpu.VMEM((ROWS_PER_W,), jnp.float32),
        ],
        compiler_params=pltpu.CompilerParams(needs_layout_passes=False),
    )
    return f(xf, cf, sf, oc, osg, qpack, pc)
